# baseline (speedup 1.0000x reference)
import functools

import jax
import jax.numpy as jnp
from jax import lax
from jax.experimental import pallas as pl
from jax.experimental.pallas import tpu as pltpu
from jax.experimental.pallas import tpu_sc as plsc

T_TC = 1792


def _halfblock(x, tgt, lens, t0):
    Th, Bb, Vb = x.shape
    m = jnp.max(x, axis=-1)
    idx = jax.lax.broadcasted_iota(jnp.int32, x.shape, 2)
    cand = jnp.where(x == m[..., None], idx, Vb)
    pred = jnp.min(cand, axis=-1)
    tids = t0 + jax.lax.broadcasted_iota(jnp.int32, (Th, Bb), 0)
    mask = tids < (lens - 1)
    corr = jnp.logical_and(pred == tgt, mask)
    c = jnp.sum(corr.astype(jnp.float32))
    return c


def _tc_body(lens_ref, x1_ref, x2_ref, tgt_ref, out_ref, acc_ref):
    i = pl.program_id(0)

    @pl.when(i == 0)
    def _init():
        acc_ref[0] = 0.0
        acc_ref[1] = jnp.sum((lens_ref[...] - 1).astype(jnp.float32))

    Th = x1_ref.shape[0]
    lens = lens_ref[...]
    tgt = tgt_ref[...]
    c1 = _halfblock(x1_ref[...], tgt[:Th], lens, i * 2 * Th)
    c2 = _halfblock(x2_ref[...], tgt[Th:], lens, i * 2 * Th + Th)
    acc_ref[0] += c1 + c2

    @pl.when(i == pl.num_programs(0) - 1)
    def _fini():
        lane = jax.lax.broadcasted_iota(jnp.int32, (1, 128), 1)
        out_ref[...] = jnp.where(
            lane == 0, acc_ref[0],
            jnp.where(lane == 1, acc_ref[1], 0.0)).astype(jnp.float32)


def _make_sc_kernel(T, B, V, t0):
    NW = 32
    n_rows = (T - t0) * B
    rpw = n_rows // NW
    G = 4
    ngroups = rpw // G
    NCH = V // 16
    mesh = plsc.VectorSubcoreMesh(core_axis_name="c", subcore_axis_name="s")

    @functools.partial(
        pl.kernel, mesh=mesh,
        out_type=jax.ShapeDtypeStruct((NW, 16), jnp.float32),
        scratch_types=[
            pltpu.VMEM((2, G, V), jnp.float32),
            pltpu.VMEM((rpw,), jnp.int32),
            pltpu.VMEM((16,), jnp.int32),
            pltpu.VMEM((16,), jnp.float32),
            pltpu.SemaphoreType.DMA,
            pltpu.SemaphoreType.DMA,
        ],
    )
    def sc_kernel(x_hbm, tgt_hbm, lens_hbm, out_hbm,
                  buf, tgt_v, lens_v, cnt_v, sem0, sem1):
        wid = lax.axis_index("s") * 2 + lax.axis_index("c")
        row0 = wid * rpw
        grow0 = t0 * B + row0

        pltpu.sync_copy(tgt_hbm.at[pl.ds(t0 * B + row0, rpw)], tgt_v)
        pltpu.sync_copy(lens_hbm, lens_v)

        sems = (sem0, sem1)
        lanes = lax.broadcasted_iota(jnp.int32, (16,), 0)

        def group_dma(g, slot):
            gr = grow0 + g * G
            src = x_hbm.at[gr // B, pl.ds(gr % B, G)]
            return pltpu.make_async_copy(src, buf.at[slot], sems[slot])

        def row_count(bufslot, q, sk):
            def chunk_body(j, carry):
                vm, vidx = carry
                x = bufslot[pl.ds(j * 16, 16)]
                upd = x > vm
                vm = jnp.where(upd, x, vm)
                vidx = jnp.where(upd, jnp.full((16,), 0, jnp.int32) + j, vidx)
                return vm, vidx

            vm0 = jnp.full((16,), -jnp.inf, jnp.float32)
            vi0 = jnp.zeros((16,), jnp.int32)
            vm, vidx = lax.fori_loop(0, NCH, chunk_body, (vm0, vi0),
                                     unroll=8)
            dnums = lax.GatherDimensionNumbers(
                offset_dims=(), collapsed_slice_dims=(0,),
                start_index_map=(0,))

            def shuffle(v, sh):
                return lax.gather(
                    v, (lanes ^ sh)[:, None], dnums, slice_sizes=(1,),
                    mode=lax.GatherScatterMode.PROMISE_IN_BOUNDS)

            m = vm
            for sh in (8, 4, 2, 1):
                m = jnp.maximum(m, shuffle(m, sh))
            gidx = vidx * 16 + lanes
            cand = jnp.where(vm == m, gidx, jnp.int32(1) << 20)
            for sh in (8, 4, 2, 1):
                cand = jnp.minimum(cand, shuffle(cand, sh))
            pred = cand[0]
            r = q * 16 + sk
            tgt_r = tgt_v[pl.ds(q * 16, 16)][sk]
            b = sk % B
            t = (grow0 + r) // B
            len_b = lens_v[...][b]
            ok = jnp.logical_and(pred == tgt_r, t < len_b - 1)
            return jnp.where(ok, 1.0, 0.0)

        group_dma(0, 0).start()

        def quad_body(q, c):
            for s in range(4):
                g = q * 4 + s
                group_dma(g, s % 2).wait()

                @pl.when(g + 1 < ngroups)
                def _():
                    group_dma(g + 1, (s + 1) % 2).start()

                for k in range(G):
                    c = c + row_count(buf.at[s % 2, k], q, s * G + k)
            return c

        c = lax.fori_loop(0, ngroups // 4, quad_body, jnp.float32(0.0))

        cnt_v[...] = jnp.where(lanes == 0, c, 0.0).astype(jnp.float32)
        pltpu.sync_copy(cnt_v, out_hbm.at[wid])

    return sc_kernel


def _combine_body(tc_ref, sc_ref, out_ref):
    c = tc_ref[0, 0] + jnp.sum(sc_ref[...])
    v = tc_ref[0, 1]
    out_ref[...] = jnp.full((1, 128), c / v, dtype=jnp.float32)


def kernel(outputs, tokens, tokens_lens):
    T, B, V = outputs.shape
    Tb = 64
    Th = Tb // 2
    n = T_TC // Tb
    targets = jnp.roll(tokens, -1, axis=0)
    lens2d = tokens_lens.reshape(1, B)
    lens16 = jnp.pad(tokens_lens, (0, 16 - B))

    tc_out = pl.pallas_call(
        _tc_body,
        grid=(n,),
        in_specs=[
            pl.BlockSpec((1, B), lambda i: (0, 0)),
            pl.BlockSpec((Th, B, V), lambda i: (2 * i, 0, 0)),
            pl.BlockSpec((Th, B, V), lambda i: (2 * i + 1, 0, 0)),
            pl.BlockSpec((Tb, B), lambda i: (i, 0)),
        ],
        out_specs=pl.BlockSpec((1, 128), lambda i: (0, 0)),
        out_shape=jax.ShapeDtypeStruct((1, 128), jnp.float32),
        scratch_shapes=[pltpu.SMEM((2,), jnp.float32)],
        compiler_params=pltpu.CompilerParams(
            dimension_semantics=("arbitrary",),
        ),
    )(lens2d, outputs, outputs, targets)

    sc_counts = _make_sc_kernel(T, B, V, T_TC)(
        outputs, targets.reshape(-1), lens16)

    acc = pl.pallas_call(
        _combine_body,
        out_shape=jax.ShapeDtypeStruct((1, 128), jnp.float32),
    )(tc_out, sc_counts)
    return acc[0, 0]


# speedup vs baseline: 1.0188x; 1.0188x over previous
import functools

import jax
import jax.numpy as jnp
from jax import lax
from jax.experimental import pallas as pl
from jax.experimental.pallas import tpu as pltpu
from jax.experimental.pallas import tpu_sc as plsc

T_TC = 1280


def _halfblock(x, tgt, lens, t0):
    Th, Bb, Vb = x.shape
    m = jnp.max(x, axis=-1)
    idx = jax.lax.broadcasted_iota(jnp.int32, x.shape, 2)
    cand = jnp.where(x == m[..., None], idx, Vb)
    pred = jnp.min(cand, axis=-1)
    tids = t0 + jax.lax.broadcasted_iota(jnp.int32, (Th, Bb), 0)
    mask = tids < (lens - 1)
    corr = jnp.logical_and(pred == tgt, mask)
    c = jnp.sum(corr.astype(jnp.float32))
    return c


def _tc_body(lens_ref, x1_ref, x2_ref, tgt_ref, out_ref, acc_ref):
    i = pl.program_id(0)

    @pl.when(i == 0)
    def _init():
        acc_ref[0] = 0.0
        acc_ref[1] = jnp.sum((lens_ref[...] - 1).astype(jnp.float32))

    Th = x1_ref.shape[0]
    lens = lens_ref[...]
    tgt = tgt_ref[...]
    c1 = _halfblock(x1_ref[...], tgt[:Th], lens, i * 2 * Th)
    c2 = _halfblock(x2_ref[...], tgt[Th:], lens, i * 2 * Th + Th)
    acc_ref[0] += c1 + c2

    @pl.when(i == pl.num_programs(0) - 1)
    def _fini():
        lane = jax.lax.broadcasted_iota(jnp.int32, (1, 128), 1)
        out_ref[...] = jnp.where(
            lane == 0, acc_ref[0],
            jnp.where(lane == 1, acc_ref[1], 0.0)).astype(jnp.float32)


def _make_sc_kernel(T, B, V, t0):
    NW = 32
    n_rows = (T - t0) * B
    rpw = n_rows // NW
    G = 4
    ngroups = rpw // G
    NCH = V // 16
    mesh = plsc.VectorSubcoreMesh(core_axis_name="c", subcore_axis_name="s")

    @functools.partial(
        pl.kernel, mesh=mesh,
        out_type=jax.ShapeDtypeStruct((NW, 16), jnp.float32),
        scratch_types=[
            pltpu.VMEM((2, G, V), jnp.float32),
            pltpu.VMEM((rpw,), jnp.int32),
            pltpu.VMEM((16,), jnp.int32),
            pltpu.VMEM((16,), jnp.float32),
            pltpu.SemaphoreType.DMA,
            pltpu.SemaphoreType.DMA,
        ],
    )
    def sc_kernel(x_hbm, tgt_hbm, lens_hbm, out_hbm,
                  buf, tgt_v, lens_v, cnt_v, sem0, sem1):
        wid = lax.axis_index("s") * 2 + lax.axis_index("c")
        row0 = wid * rpw
        grow0 = t0 * B + row0

        pltpu.sync_copy(tgt_hbm.at[pl.ds(t0 * B + row0, rpw)], tgt_v)
        pltpu.sync_copy(lens_hbm, lens_v)

        sems = (sem0, sem1)
        lanes = lax.broadcasted_iota(jnp.int32, (16,), 0)

        def group_dma(g, slot):
            gr = grow0 + g * G
            src = x_hbm.at[gr // B, pl.ds(gr % B, G)]
            return pltpu.make_async_copy(src, buf.at[slot], sems[slot])

        def row_count(bufslot, q, sk):
            def chunk_body(j, carry):
                vm, vidx = carry
                x = bufslot[pl.ds(j * 16, 16)]
                upd = x > vm
                vm = jnp.where(upd, x, vm)
                vidx = jnp.where(upd, jnp.full((16,), 0, jnp.int32) + j, vidx)
                return vm, vidx

            vm0 = jnp.full((16,), -jnp.inf, jnp.float32)
            vi0 = jnp.zeros((16,), jnp.int32)
            vm, vidx = lax.fori_loop(0, NCH, chunk_body, (vm0, vi0),
                                     unroll=8)
            dnums = lax.GatherDimensionNumbers(
                offset_dims=(), collapsed_slice_dims=(0,),
                start_index_map=(0,))

            def shuffle(v, sh):
                return lax.gather(
                    v, (lanes ^ sh)[:, None], dnums, slice_sizes=(1,),
                    mode=lax.GatherScatterMode.PROMISE_IN_BOUNDS)

            m = vm
            for sh in (8, 4, 2, 1):
                m = jnp.maximum(m, shuffle(m, sh))
            gidx = vidx * 16 + lanes
            cand = jnp.where(vm == m, gidx, jnp.int32(1) << 20)
            for sh in (8, 4, 2, 1):
                cand = jnp.minimum(cand, shuffle(cand, sh))
            pred = cand[0]
            r = q * 16 + sk
            tgt_r = tgt_v[pl.ds(q * 16, 16)][sk]
            b = sk % B
            t = (grow0 + r) // B
            len_b = lens_v[...][b]
            ok = jnp.logical_and(pred == tgt_r, t < len_b - 1)
            return jnp.where(ok, 1.0, 0.0)

        group_dma(0, 0).start()

        def quad_body(q, c):
            for s in range(4):
                g = q * 4 + s
                group_dma(g, s % 2).wait()

                @pl.when(g + 1 < ngroups)
                def _():
                    group_dma(g + 1, (s + 1) % 2).start()

                for k in range(G):
                    c = c + row_count(buf.at[s % 2, k], q, s * G + k)
            return c

        c = lax.fori_loop(0, ngroups // 4, quad_body, jnp.float32(0.0))

        cnt_v[...] = jnp.where(lanes == 0, c, 0.0).astype(jnp.float32)
        pltpu.sync_copy(cnt_v, out_hbm.at[wid])

    return sc_kernel


def _combine_body(tc_ref, sc_ref, out_ref):
    c = tc_ref[0, 0] + jnp.sum(sc_ref[...])
    v = tc_ref[0, 1]
    out_ref[...] = jnp.full((1, 128), c / v, dtype=jnp.float32)


def kernel(outputs, tokens, tokens_lens):
    T, B, V = outputs.shape
    Tb = 64
    Th = Tb // 2
    n = T_TC // Tb
    targets = jnp.roll(tokens, -1, axis=0)
    lens2d = tokens_lens.reshape(1, B)
    lens16 = jnp.pad(tokens_lens, (0, 16 - B))

    tc_out = pl.pallas_call(
        _tc_body,
        grid=(n,),
        in_specs=[
            pl.BlockSpec((1, B), lambda i: (0, 0)),
            pl.BlockSpec((Th, B, V), lambda i: (2 * i, 0, 0)),
            pl.BlockSpec((Th, B, V), lambda i: (2 * i + 1, 0, 0)),
            pl.BlockSpec((Tb, B), lambda i: (i, 0)),
        ],
        out_specs=pl.BlockSpec((1, 128), lambda i: (0, 0)),
        out_shape=jax.ShapeDtypeStruct((1, 128), jnp.float32),
        scratch_shapes=[pltpu.SMEM((2,), jnp.float32)],
        compiler_params=pltpu.CompilerParams(
            dimension_semantics=("arbitrary",),
        ),
    )(lens2d, outputs, outputs, targets)

    sc_counts = _make_sc_kernel(T, B, V, T_TC)(
        outputs, targets.reshape(-1), lens16)

    acc = pl.pallas_call(
        _combine_body,
        out_shape=jax.ShapeDtypeStruct((1, 128), jnp.float32),
    )(tc_out, sc_counts)
    return acc[0, 0]


# speedup vs baseline: 1.1534x; 1.1321x over previous
import jax
import jax.numpy as jnp
from jax.experimental import pallas as pl
from jax.experimental.pallas import tpu as pltpu


def _halfblock(x, tgt, lens, t0):
    Th, Bb, Vb = x.shape
    m = jnp.max(x, axis=-1)
    idx = jax.lax.broadcasted_iota(jnp.int32, x.shape, 2)
    cand = jnp.where(x == m[..., None], idx, Vb)
    pred = jnp.min(cand, axis=-1)
    tids = t0 + jax.lax.broadcasted_iota(jnp.int32, (Th, Bb), 0)
    mask = tids < (lens - 1)
    corr = jnp.logical_and(pred == tgt, mask)
    c = jnp.sum(corr.astype(jnp.float32))
    v = jnp.sum(mask.astype(jnp.float32))
    return c, v


def _body(lens_ref, x1_ref, x2_ref, tok_ref, nxt_ref, out_ref, acc_ref):
    i = pl.program_id(0)

    @pl.when(i == 0)
    def _init():
        acc_ref[0] = 0.0
        acc_ref[1] = 0.0

    Th = x1_ref.shape[0]
    lens = lens_ref[...]
    tgt = jnp.concatenate([tok_ref[1:], nxt_ref[:1]], axis=0)
    c1, v1 = _halfblock(x1_ref[...], tgt[:Th], lens, i * 2 * Th)
    c2, v2 = _halfblock(x2_ref[...], tgt[Th:], lens, i * 2 * Th + Th)
    acc_ref[0] += c1 + c2
    acc_ref[1] += v1 + v2

    @pl.when(i == pl.num_programs(0) - 1)
    def _fini():
        out_ref[...] = jnp.full((1, 128), acc_ref[0] / acc_ref[1],
                                dtype=jnp.float32)


def kernel(outputs, tokens, tokens_lens):
    T, B, V = outputs.shape
    Tb = 64
    Th = Tb // 2
    n = T // Tb
    lens2d = tokens_lens.reshape(1, B)

    acc = pl.pallas_call(
        _body,
        grid=(n,),
        in_specs=[
            pl.BlockSpec((1, B), lambda i: (0, 0)),
            pl.BlockSpec((Th, B, V), lambda i: (2 * i, 0, 0)),
            pl.BlockSpec((Th, B, V), lambda i: (2 * i + 1, 0, 0)),
            pl.BlockSpec((Tb, B), lambda i: (i, 0)),
            pl.BlockSpec((Tb, B), lambda i: (jnp.minimum(i + 1, n - 1), 0)),
        ],
        out_specs=pl.BlockSpec((1, 128), lambda i: (0, 0)),
        out_shape=jax.ShapeDtypeStruct((1, 128), jnp.float32),
        scratch_shapes=[pltpu.SMEM((2,), jnp.float32)],
        compiler_params=pltpu.CompilerParams(
            dimension_semantics=("arbitrary",),
        ),
    )(lens2d, outputs, outputs, tokens, tokens)
    return acc[0, 0]


# speedup vs baseline: 1.1647x; 1.0098x over previous
import jax
import jax.numpy as jnp
from jax.experimental import pallas as pl
from jax.experimental.pallas import tpu as pltpu


def _halfblock(x, tgt, lens, t0):
    Th, Bb, Vb = x.shape
    m = jnp.max(x, axis=-1)
    idx = jax.lax.broadcasted_iota(jnp.int32, x.shape, 2).astype(jnp.float32)
    cand = jnp.where(x == m[..., None], idx, float(Vb))
    pred = jnp.min(cand, axis=-1)
    tids = t0 + jax.lax.broadcasted_iota(jnp.int32, (Th, Bb), 0)
    mask = tids < (lens - 1)
    corr = jnp.logical_and(pred == tgt.astype(jnp.float32), mask)
    c = jnp.sum(corr.astype(jnp.float32))
    v = jnp.sum(mask.astype(jnp.float32))
    return c, v


def _body(lens_ref, x1_ref, x2_ref, tok_ref, nxt_ref, out_ref, acc_ref):
    i = pl.program_id(0)

    @pl.when(i == 0)
    def _init():
        acc_ref[0] = 0.0
        acc_ref[1] = 0.0

    Th = x1_ref.shape[0]
    lens = lens_ref[...]
    tgt = jnp.concatenate([tok_ref[1:], nxt_ref[:1]], axis=0)
    c1, v1 = _halfblock(x1_ref[...], tgt[:Th], lens, i * 2 * Th)
    c2, v2 = _halfblock(x2_ref[...], tgt[Th:], lens, i * 2 * Th + Th)
    acc_ref[0] += c1 + c2
    acc_ref[1] += v1 + v2

    @pl.when(i == pl.num_programs(0) - 1)
    def _fini():
        out_ref[...] = jnp.full((1, 128), acc_ref[0] / acc_ref[1],
                                dtype=jnp.float32)


def kernel(outputs, tokens, tokens_lens):
    T, B, V = outputs.shape
    Tb = 64
    Th = Tb // 2
    n = T // Tb
    lens2d = tokens_lens.reshape(1, B)

    acc = pl.pallas_call(
        _body,
        grid=(n,),
        in_specs=[
            pl.BlockSpec((1, B), lambda i: (0, 0)),
            pl.BlockSpec((Th, B, V), lambda i: (2 * i, 0, 0)),
            pl.BlockSpec((Th, B, V), lambda i: (2 * i + 1, 0, 0)),
            pl.BlockSpec((Tb, B), lambda i: (i, 0)),
            pl.BlockSpec((Tb, B), lambda i: (jnp.minimum(i + 1, n - 1), 0)),
        ],
        out_specs=pl.BlockSpec((1, 128), lambda i: (0, 0)),
        out_shape=jax.ShapeDtypeStruct((1, 128), jnp.float32),
        scratch_shapes=[pltpu.SMEM((2,), jnp.float32)],
        compiler_params=pltpu.CompilerParams(
            dimension_semantics=("arbitrary",),
        ),
    )(lens2d, outputs, outputs, tokens, tokens)
    return acc[0, 0]
